# baseline (speedup 1.0000x reference)
import functools

import jax
import jax.numpy as jnp
from jax import lax
from jax.experimental import pallas as pl
from jax.experimental.pallas import tpu as pltpu
from jax.experimental.pallas import tpu_sc as plsc

NUM_ENTITIES = 1000000
EMBED_DIM = 64
BATCH = 16384

NC = 2
NS = 16
NW = NC * NS
B_PER_W = BATCH // NW
CHUNK = 128
NCHUNK = B_PER_W // CHUNK


def _sc_kernel(src_hbm, tgt_hbm, emb_hbm, rel_hbm, out_hbm,
               sidx, tidx, srows, trows, relv, outv, outs, sem):
    cid = lax.axis_index("c")
    sid = lax.axis_index("s")
    wid = sid * NC + cid
    base = wid * B_PER_W

    pltpu.sync_copy(rel_hbm, relv)
    for j in range(NCHUNK):
        pltpu.sync_copy(src_hbm.at[pl.ds(base + j * CHUNK, CHUNK)], sidx.at[j])
        pltpu.sync_copy(tgt_hbm.at[pl.ds(base + j * CHUNK, CHUNK)], tidx.at[j])

    handles = []
    for j in range(NCHUNK):
        handles.append(pltpu.async_copy(emb_hbm.at[sidx.at[j]], srows.at[j], sem))
        handles.append(pltpu.async_copy(emb_hbm.at[tidx.at[j]], trows.at[j], sem))
    for h in handles:
        h.wait()

    rel_q = [relv[0, pl.ds(q * 16, 16)] for q in range(EMBED_DIM // 16)]

    for j in range(NCHUNK):
        def row_body(i, _, j=j):
            acc = None
            for q in range(EMBED_DIM // 16):
                s = srows[j, i, pl.ds(q * 16, 16)]
                t = trows[j, i, pl.ds(q * 16, 16)]
                d = jnp.abs(s - t + rel_q[q])
                acc = d if acc is None else acc + d
            outs[j * CHUNK + i] = jnp.sum(acc)
            return 0
        lax.fori_loop(0, CHUNK, row_body, 0)

    lanes = lax.iota(jnp.int32, 16)

    def grp_body(g, _):
        v = jnp.zeros((16,), jnp.float32)
        for r in range(16):
            v = jnp.where(lanes == r, outs[g * 16 + r], v)
        outv[pl.ds(g * 16, 16)] = v
        return 0

    lax.fori_loop(0, B_PER_W // 16, grp_body, 0)

    pltpu.sync_copy(outv, out_hbm.at[pl.ds(base, B_PER_W)])


@jax.jit
def _transe_score(sources, targets, entity_emb, relation_emb):
    mesh = plsc.VectorSubcoreMesh(core_axis_name="c", subcore_axis_name="s")
    kern = functools.partial(
        pl.kernel,
        out_type=jax.ShapeDtypeStruct((BATCH,), jnp.float32),
        mesh=mesh,
        compiler_params=pltpu.CompilerParams(needs_layout_passes=False,
                                             use_tc_tiling_on_sc=False),
        scratch_types=[
            pltpu.VMEM((NCHUNK, CHUNK), jnp.int32),
            pltpu.VMEM((NCHUNK, CHUNK), jnp.int32),
            pltpu.VMEM((NCHUNK, CHUNK, EMBED_DIM), jnp.float32),
            pltpu.VMEM((NCHUNK, CHUNK, EMBED_DIM), jnp.float32),
            pltpu.VMEM((1, EMBED_DIM), jnp.float32),
            pltpu.VMEM((B_PER_W,), jnp.float32),
            pltpu.SMEM((B_PER_W,), jnp.float32),
            pltpu.SemaphoreType.DMA,
        ],
    )(_sc_kernel)
    return kern(sources, targets, entity_emb, relation_emb)


def kernel(sources, targets, entity_emb, relation_emb):
    return _transe_score(sources.astype(jnp.int32), targets.astype(jnp.int32),
                         entity_emb, relation_emb)
